# baseline (speedup 1.0000x reference)
import functools

import jax
import jax.numpy as jnp
from jax import lax
from jax.experimental import pallas as pl
from jax.experimental.pallas import tpu as pltpu
from jax.experimental.pallas import tpu_sc as plsc

NC = 2
NS = 16
NW = NC * NS
CH = 64


def _mm(a, b):
    return jax.lax.dot_general(a, b, (((1,), (0,)), ((), ())),
                               precision=jax.lax.Precision.HIGHEST,
                               preferred_element_type=jnp.float32)


def _fold_body(w1t_ref, wrelt_ref, wroott_ref, brel_ref, b1_ref,
               mut_ref, mvt_ref, c_ref):
    w1t = w1t_ref[...]
    mut_ref[...] = _mm(wrelt_ref[...], w1t)
    mvt_ref[...] = _mm(wroott_ref[...], w1t)
    c_ref[...] = _mm(brel_ref[...], w1t) + b1_ref[...]


def _uv_body(x_ref, mut_ref, mvt_ref, u_ref, v_ref):
    xb = x_ref[...]
    u_ref[...] = _mm(xb, mut_ref[...])
    v_ref[...] = _mm(xb, mvt_ref[...])


def _head_body(a0_ref, a1_ref, v_ref, c_ref, w2_ref, b2_ref, out_ref):
    z = jnp.maximum(a0_ref[...] + a1_ref[...] + v_ref[...] + c_ref[...], 0.0)
    out_ref[...] = jnp.sum(z * w2_ref[...], axis=1) + b2_ref[0, 0]


NBUF = 2
RING = 4


def _make_sc_scatter(N_pad, H, NCHUNK):
    mesh = plsc.VectorSubcoreMesh(core_axis_name="c", subcore_axis_name="s",
                                  num_cores=NC, num_subcores=NS)
    rows_per_tile = N_pad // NS

    @functools.partial(
        pl.kernel,
        out_type=jax.ShapeDtypeStruct((NC, N_pad, H), jnp.float32),
        mesh=mesh,
        scratch_types=[
            pltpu.VMEM((NCHUNK + NBUF, CH), jnp.int32),
            pltpu.VMEM((NCHUNK, CH), jnp.int32),
            pltpu.VMEM((RING, CH, H), jnp.float32),
            pltpu.VMEM_SHARED((N_pad, H), jnp.float32),
        ] + [pltpu.SemaphoreType.DMA] * (2 * RING),
        compiler_params=pltpu.CompilerParams(use_tc_tiling_on_sc=False),
    )
    def sc_scatter(u_hbm, srcw_hbm, dstw_hbm, zeros_hbm, out_hbm,
                   src_v, dst_v, rows_v, agg_sh, *sems):
        gsems = sems[:RING]
        ssems = sems[RING:]
        cid = lax.axis_index("c")
        sid = lax.axis_index("s")
        wid = sid * NC + cid
        off = sid * rows_per_tile
        pltpu.sync_copy(zeros_hbm.at[pl.ds(off, rows_per_tile)],
                        agg_sh.at[pl.ds(off, rows_per_tile)])
        pltpu.sync_copy(srcw_hbm.at[wid], src_v)
        pltpu.sync_copy(dstw_hbm.at[wid], dst_v)
        plsc.subcore_barrier()
        pltpu.async_copy(u_hbm.at[src_v.at[0]], rows_v.at[0], gsems[0])
        pltpu.async_copy(u_hbm.at[src_v.at[1]], rows_v.at[1], gsems[1])

        def body(r, carry):
            for b in range(RING):
                i = r * RING + b
                b2 = (b + 2) % RING
                pltpu.make_async_copy(u_hbm.at[src_v.at[0]], rows_v.at[b],
                                      gsems[b]).wait()
                pltpu.async_copy(rows_v.at[b], agg_sh.at[dst_v.at[i]],
                                 ssems[b], add=True)

                @pl.when(jnp.logical_or(r > 0, b >= 2))
                def _():
                    pltpu.make_async_copy(rows_v.at[b2],
                                          agg_sh.at[dst_v.at[0]],
                                          ssems[b2]).wait()

                pltpu.async_copy(u_hbm.at[src_v.at[i + 2]], rows_v.at[b2],
                                 gsems[b2])
            return carry

        lax.fori_loop(0, NCHUNK // RING, body, 0)
        for b in (2, 3):
            pltpu.make_async_copy(rows_v.at[b], agg_sh.at[dst_v.at[0]],
                                  ssems[b]).wait()
        for b in (0, 1):
            pltpu.make_async_copy(u_hbm.at[src_v.at[0]], rows_v.at[b],
                                  gsems[b]).wait()
        plsc.subcore_barrier()
        pltpu.sync_copy(agg_sh.at[pl.ds(off, rows_per_tile)],
                        out_hbm.at[cid, pl.ds(off, rows_per_tile)])

    return sc_scatter


def kernel(x, edge_index, W_rel, b_rel, W_root, W1, b1, W2, b2):
    N, D = x.shape
    E = edge_index.shape[1]
    H = W1.shape[0]

    N_pad = 10240
    BN = 2048
    NCHUNK = -(-E // (NW * CH))
    NCHUNK = -(-NCHUNK // RING) * RING
    E_pad = NW * CH * NCHUNK

    brel2 = b_rel.reshape(1, D)
    b12 = b1.reshape(1, H)
    mut, mvt, c = pl.pallas_call(
        _fold_body,
        out_shape=(jax.ShapeDtypeStruct((D, H), jnp.float32),
                   jax.ShapeDtypeStruct((D, H), jnp.float32),
                   jax.ShapeDtypeStruct((1, H), jnp.float32)),
    )(W1.T, W_rel.T, W_root.T, brel2, b12)

    x_pad = jnp.zeros((N_pad, D), jnp.float32).at[:N].set(x)
    grid = (N_pad // BN,)
    u, v = pl.pallas_call(
        _uv_body,
        grid=grid,
        in_specs=[
            pl.BlockSpec((BN, D), lambda i: (i, 0)),
            pl.BlockSpec((D, H), lambda i: (0, 0)),
            pl.BlockSpec((D, H), lambda i: (0, 0)),
        ],
        out_specs=(pl.BlockSpec((BN, H), lambda i: (i, 0)),
                   pl.BlockSpec((BN, H), lambda i: (i, 0))),
        out_shape=(jax.ShapeDtypeStruct((N_pad, H), jnp.float32),
                   jax.ShapeDtypeStruct((N_pad, H), jnp.float32)),
    )(x_pad, mut, mvt)

    src = edge_index[0].astype(jnp.int32)
    dst = edge_index[1].astype(jnp.int32)
    src_p = jnp.full((E_pad,), N, jnp.int32).at[:E].set(src)
    dst_p = jnp.full((E_pad,), N, jnp.int32).at[:E].set(dst)
    srcw = src_p.reshape(NW, NCHUNK, CH)
    srcw = jnp.concatenate(
        [srcw, jnp.zeros((NW, NBUF, CH), jnp.int32)], axis=1)
    dstw = dst_p.reshape(NW, NCHUNK, CH)
    zeros_hbm = jnp.zeros((N_pad, H), jnp.float32)

    aggp = _make_sc_scatter(N_pad, H, NCHUNK)(u, srcw, dstw, zeros_hbm)

    b22 = b2.reshape(1, 1)
    out1d = pl.pallas_call(
        _head_body,
        grid=grid,
        in_specs=[
            pl.BlockSpec((BN, H), lambda i: (i, 0)),
            pl.BlockSpec((BN, H), lambda i: (i, 0)),
            pl.BlockSpec((BN, H), lambda i: (i, 0)),
            pl.BlockSpec((1, H), lambda i: (0, 0)),
            pl.BlockSpec((1, H), lambda i: (0, 0)),
            pl.BlockSpec((1, 1), lambda i: (0, 0)),
        ],
        out_specs=pl.BlockSpec((BN,), lambda i: (i,)),
        out_shape=jax.ShapeDtypeStruct((N_pad,), jnp.float32),
    )(aggp[0], aggp[1], v, c, W2, b22)

    return out1d[:N]


# speedup vs baseline: 1.0762x; 1.0762x over previous
import functools

import jax
import jax.numpy as jnp
from jax import lax
from jax.experimental import pallas as pl
from jax.experimental.pallas import tpu as pltpu
from jax.experimental.pallas import tpu_sc as plsc

NC = 2
NS = 16
NW = NC * NS
CH = 128


def _mm(a, b):
    return jax.lax.dot_general(a, b, (((1,), (0,)), ((), ())),
                               precision=jax.lax.Precision.HIGHEST,
                               preferred_element_type=jnp.float32)


def _fold_body(w1t_ref, wrelt_ref, wroott_ref, brel_ref, b1_ref,
               mut_ref, mvt_ref, c_ref):
    w1t = w1t_ref[...]
    mut_ref[...] = _mm(wrelt_ref[...], w1t)
    mvt_ref[...] = _mm(wroott_ref[...], w1t)
    c_ref[...] = _mm(brel_ref[...], w1t) + b1_ref[...]


def _uv_body(x_ref, mut_ref, mvt_ref, u_ref, v_ref):
    xb = x_ref[...]
    u_ref[...] = _mm(xb, mut_ref[...])
    v_ref[...] = _mm(xb, mvt_ref[...])


def _head_body(a0_ref, a1_ref, v_ref, c_ref, w2_ref, b2_ref, out_ref):
    z = jnp.maximum(a0_ref[...] + a1_ref[...] + v_ref[...] + c_ref[...], 0.0)
    out_ref[...] = jnp.sum(z * w2_ref[...], axis=1) + b2_ref[0, 0]


NBUF = 2
RING = 4


def _make_sc_scatter(N_pad, H, NCHUNK):
    mesh = plsc.VectorSubcoreMesh(core_axis_name="c", subcore_axis_name="s",
                                  num_cores=NC, num_subcores=NS)
    rows_per_tile = N_pad // NS

    @functools.partial(
        pl.kernel,
        out_type=jax.ShapeDtypeStruct((NC, N_pad, H), jnp.float32),
        mesh=mesh,
        scratch_types=[
            pltpu.VMEM((NCHUNK + NBUF, CH), jnp.int32),
            pltpu.VMEM((NCHUNK, CH), jnp.int32),
            pltpu.VMEM((RING, CH, H), jnp.float32),
            pltpu.VMEM_SHARED((N_pad, H), jnp.float32),
        ] + [pltpu.SemaphoreType.DMA] * (2 * RING),
        compiler_params=pltpu.CompilerParams(use_tc_tiling_on_sc=False),
    )
    def sc_scatter(u_hbm, srcw_hbm, dstw_hbm, zeros_hbm, out_hbm,
                   src_v, dst_v, rows_v, agg_sh, *sems):
        gsems = sems[:RING]
        ssems = sems[RING:]
        cid = lax.axis_index("c")
        sid = lax.axis_index("s")
        wid = sid * NC + cid
        off = sid * rows_per_tile
        pltpu.sync_copy(srcw_hbm.at[pl.ds(wid * NCHUNK, NCHUNK)],
                        src_v.at[pl.ds(0, NCHUNK)])
        pltpu.async_copy(u_hbm.at[src_v.at[0]], rows_v.at[0], gsems[0])
        pltpu.async_copy(u_hbm.at[src_v.at[1]], rows_v.at[1], gsems[1])
        for r_ in range(NBUF):
            for k_ in range(CH // 16):
                src_v[NCHUNK + r_, pl.ds(16 * k_, 16)] = jnp.zeros(
                    (16,), jnp.int32)
        pltpu.sync_copy(dstw_hbm.at[pl.ds(wid * NCHUNK, NCHUNK)], dst_v)
        pltpu.sync_copy(zeros_hbm.at[pl.ds(off, rows_per_tile)],
                        agg_sh.at[pl.ds(off, rows_per_tile)])
        plsc.subcore_barrier()

        def body(r, carry):
            for b in range(RING):
                i = r * RING + b
                b2 = (b + 2) % RING
                pltpu.make_async_copy(u_hbm.at[src_v.at[0]], rows_v.at[b],
                                      gsems[b]).wait()
                pltpu.async_copy(rows_v.at[b], agg_sh.at[dst_v.at[i]],
                                 ssems[b], add=True)

                @pl.when(jnp.logical_or(r > 0, b >= 2))
                def _():
                    pltpu.make_async_copy(rows_v.at[b2],
                                          agg_sh.at[dst_v.at[0]],
                                          ssems[b2]).wait()

                pltpu.async_copy(u_hbm.at[src_v.at[i + 2]], rows_v.at[b2],
                                 gsems[b2])
            return carry

        lax.fori_loop(0, NCHUNK // RING, body, 0)
        for b in (2, 3):
            pltpu.make_async_copy(rows_v.at[b], agg_sh.at[dst_v.at[0]],
                                  ssems[b]).wait()
        for b in (0, 1):
            pltpu.make_async_copy(u_hbm.at[src_v.at[0]], rows_v.at[b],
                                  gsems[b]).wait()
        plsc.subcore_barrier()
        pltpu.sync_copy(agg_sh.at[pl.ds(off, rows_per_tile)],
                        out_hbm.at[cid, pl.ds(off, rows_per_tile)])

    return sc_scatter


def kernel(x, edge_index, W_rel, b_rel, W_root, W1, b1, W2, b2):
    N, D = x.shape
    E = edge_index.shape[1]
    H = W1.shape[0]

    N_pad = 10240
    BN = 2048
    NCHUNK = -(-E // (NW * CH))
    NCHUNK = -(-NCHUNK // RING) * RING
    E_pad = NW * CH * NCHUNK

    brel2 = b_rel.reshape(1, D)
    b12 = b1.reshape(1, H)
    mut, mvt, c = pl.pallas_call(
        _fold_body,
        out_shape=(jax.ShapeDtypeStruct((D, H), jnp.float32),
                   jax.ShapeDtypeStruct((D, H), jnp.float32),
                   jax.ShapeDtypeStruct((1, H), jnp.float32)),
    )(W1.T, W_rel.T, W_root.T, brel2, b12)

    x_pad = jnp.zeros((N_pad, D), jnp.float32).at[:N].set(x)
    grid = (N_pad // BN,)
    u, v = pl.pallas_call(
        _uv_body,
        grid=grid,
        in_specs=[
            pl.BlockSpec((BN, D), lambda i: (i, 0)),
            pl.BlockSpec((D, H), lambda i: (0, 0)),
            pl.BlockSpec((D, H), lambda i: (0, 0)),
        ],
        out_specs=(pl.BlockSpec((BN, H), lambda i: (i, 0)),
                   pl.BlockSpec((BN, H), lambda i: (i, 0))),
        out_shape=(jax.ShapeDtypeStruct((N_pad, H), jnp.float32),
                   jax.ShapeDtypeStruct((N_pad, H), jnp.float32)),
    )(x_pad, mut, mvt)

    src = edge_index[0].astype(jnp.int32)
    dst = edge_index[1].astype(jnp.int32)
    src_p = jnp.full((E_pad,), N, jnp.int32).at[:E].set(src)
    dst_p = jnp.full((E_pad,), N, jnp.int32).at[:E].set(dst)
    srcw = src_p.reshape(NW * NCHUNK, CH)
    dstw = dst_p.reshape(NW * NCHUNK, CH)
    zeros_hbm = jnp.zeros((N_pad, H), jnp.float32)

    aggp = _make_sc_scatter(N_pad, H, NCHUNK)(u, srcw, dstw, zeros_hbm)

    b22 = b2.reshape(1, 1)
    out1d = pl.pallas_call(
        _head_body,
        grid=grid,
        in_specs=[
            pl.BlockSpec((BN, H), lambda i: (i, 0)),
            pl.BlockSpec((BN, H), lambda i: (i, 0)),
            pl.BlockSpec((BN, H), lambda i: (i, 0)),
            pl.BlockSpec((1, H), lambda i: (0, 0)),
            pl.BlockSpec((1, H), lambda i: (0, 0)),
            pl.BlockSpec((1, 1), lambda i: (0, 0)),
        ],
        out_specs=pl.BlockSpec((BN,), lambda i: (i,)),
        out_shape=jax.ShapeDtypeStruct((N_pad,), jnp.float32),
    )(aggp[0], aggp[1], v, c, W2, b22)

    return out1d[:N]
